# baseline (speedup 1.0000x reference)
import jax
import jax.numpy as jnp
from jax.experimental import pallas as pl
from jax.experimental.pallas import tpu as pltpu

_GROUP = 64
_TM_DENSE = 2048
_TM_FALLBACK = 16384


def _dense_matmul_kernel(xg_ref, b_ref, o_ref):
    o_ref[...] = jnp.dot(
        xg_ref[...], b_ref[...], preferred_element_type=jnp.float32
    )


def _rowtile_kernel(x_ref, w0_ref, w1_ref, o_ref):
    w = jnp.dot(w0_ref[...], w1_ref[...], preferred_element_type=jnp.float32)
    o_ref[...] = jnp.dot(x_ref[...], w, preferred_element_type=jnp.float32)


def _rowtile_path(x, pl0, weight1):
    n, k = x.shape
    n_out = weight1.shape[1]
    tm = min(_TM_FALLBACK, n)
    return pl.pallas_call(
        _rowtile_kernel,
        out_shape=jax.ShapeDtypeStruct((n, n_out), x.dtype),
        grid=(pl.cdiv(n, tm),),
        in_specs=[
            pl.BlockSpec((tm, k), lambda i: (i, 0)),
            pl.BlockSpec((k, pl0.shape[1]), lambda i: (0, 0)),
            pl.BlockSpec((weight1.shape[0], n_out), lambda i: (0, 0)),
        ],
        out_specs=pl.BlockSpec((tm, n_out), lambda i: (i, 0)),
        compiler_params=pltpu.CompilerParams(
            dimension_semantics=("parallel",),
            vmem_limit_bytes=100 << 20,
        ),
    )(x, pl0, weight1)


def kernel(x, pl0, pl1, weight1, weight2):
    n, k = x.shape
    n_out = weight1.shape[1]
    if n % _GROUP or k != 10 or n_out != 10:
        return _rowtile_path(x, pl0, weight1)

    w = jnp.dot(pl0, weight1, preferred_element_type=jnp.float32)
    b = jnp.kron(jnp.eye(_GROUP, dtype=jnp.float32), w)

    m = n // _GROUP
    d = _GROUP * k
    xg = x.reshape(m, d)
    tm = min(_TM_DENSE, m)
    cost = pl.CostEstimate(
        flops=2 * m * d * d,
        transcendentals=0,
        bytes_accessed=(2 * m * d + d * d) * 4,
    )
    out = pl.pallas_call(
        _dense_matmul_kernel,
        out_shape=jax.ShapeDtypeStruct((m, d), jnp.float32),
        grid=(pl.cdiv(m, tm),),
        in_specs=[
            pl.BlockSpec((tm, d), lambda i: (i, 0)),
            pl.BlockSpec((d, d), lambda i: (0, 0)),
        ],
        out_specs=pl.BlockSpec((tm, d), lambda i: (i, 0)),
        compiler_params=pltpu.CompilerParams(
            dimension_semantics=("parallel",),
            vmem_limit_bytes=100 << 20,
        ),
        cost_estimate=cost,
    )(xg, b)
    return out.reshape(n, n_out)
